# Initial kernel scaffold; baseline (speedup 1.0000x reference)
import jax
import jax.numpy as jnp
from jax.experimental import pallas as pl


def kernel(x, edge_index, labels, inferenz_anchors, W1, b1, W2, b2):
    raise NotImplementedError("write your pallas kernel here")


# speedup vs baseline: 10.4267x; 10.4267x over previous
import functools

import jax
import jax.numpy as jnp
from jax import lax
from jax.experimental import pallas as pl
from jax.experimental.pallas import tpu as pltpu
from jax.experimental.pallas import tpu_sc as plsc

N_NODES = 10000
N_PAD = 10240
CHUNK = 128
NW = 32
E_EDGES = 320000
N_CHUNKS = -(-E_EDGES // (CHUNK * NW)) * NW
E_PAD = N_CHUNKS * CHUNK
CPW = N_CHUNKS // NW
ROWS_PER_TILE = N_PAD // 16

def _mesh():
    return plsc.VectorSubcoreMesh(core_axis_name="c", subcore_axis_name="s")


def _zero_vmem(ref, n_rows, n16):
    z = jnp.zeros((16,), jnp.float32)

    def body(i, _):
        for j in range(n16):
            ref[i, pl.ds(j * 16, 16)] = z
        return 0

    lax.fori_loop(0, n_rows, body, 0)


def _deg_kernel(dst_hbm, out_hbm, idx_v, ones_v, zeros_v, acc_sh, sem):
    cid = lax.axis_index("c")
    sid = lax.axis_index("s")
    wid = sid * 2 + cid

    one = jnp.full((16,), 1.0, jnp.float32)
    zero = jnp.zeros((16,), jnp.float32)

    def init_body(i, _):
        ones_v[pl.ds(i * 16, 16)] = one
        return 0

    lax.fori_loop(0, CHUNK // 16, init_body, 0)

    def zero_body(i, _):
        zeros_v[pl.ds(i * 16, 16)] = zero
        return 0

    lax.fori_loop(0, ROWS_PER_TILE // 16, zero_body, 0)

    pltpu.sync_copy(zeros_v, acc_sh.at[pl.ds(sid * ROWS_PER_TILE, ROWS_PER_TILE)])
    plsc.subcore_barrier()

    def chunk_body(j, _):
        off = (wid * CPW + j) * CHUNK
        pltpu.sync_copy(dst_hbm.at[pl.ds(off, CHUNK)], idx_v)
        pltpu.sync_copy(ones_v, acc_sh.at[idx_v], add=True)
        return 0

    lax.fori_loop(0, CPW, chunk_body, 0)
    plsc.subcore_barrier()

    pltpu.sync_copy(acc_sh.at[pl.ds(sid * ROWS_PER_TILE, ROWS_PER_TILE)],
                    out_hbm.at[cid, pl.ds(sid * ROWS_PER_TILE, ROWS_PER_TILE)])


def _spmm_kernel(F, g_hbm, src_hbm, dst_hbm, out_hbm,
                 sidx_v, didx_v, rows_v, acc_sh, sem):
    cid = lax.axis_index("c")
    sid = lax.axis_index("s")
    wid = sid * 2 + cid

    _zero_vmem(rows_v, CHUNK, F // 16)
    for k in range(ROWS_PER_TILE // CHUNK):
        pltpu.sync_copy(
            rows_v, acc_sh.at[pl.ds(sid * ROWS_PER_TILE + k * CHUNK, CHUNK)])
    plsc.subcore_barrier()

    def chunk_body(j, _):
        off = (wid * CPW + j) * CHUNK
        pltpu.sync_copy(src_hbm.at[pl.ds(off, CHUNK)], sidx_v)
        pltpu.sync_copy(dst_hbm.at[pl.ds(off, CHUNK)], didx_v)
        pltpu.async_copy(g_hbm.at[sidx_v], rows_v, sem).wait()
        pltpu.sync_copy(rows_v, acc_sh.at[didx_v], add=True)
        return 0

    lax.fori_loop(0, CPW, chunk_body, 0)
    plsc.subcore_barrier()

    pltpu.sync_copy(acc_sh.at[pl.ds(sid * ROWS_PER_TILE, ROWS_PER_TILE)],
                    out_hbm.at[cid, pl.ds(sid * ROWS_PER_TILE, ROWS_PER_TILE)])


def _make_deg():
    return pl.kernel(
        _deg_kernel,
        mesh=_mesh(),
        out_type=jax.ShapeDtypeStruct((2, N_PAD), jnp.float32),
        scratch_types=[
            pltpu.VMEM((CHUNK,), jnp.int32),
            pltpu.VMEM((CHUNK,), jnp.float32),
            pltpu.VMEM((ROWS_PER_TILE,), jnp.float32),
            pltpu.VMEM_SHARED((N_PAD,), jnp.float32),
            pltpu.SemaphoreType.DMA,
        ],
    )


def _make_spmm(F):
    return pl.kernel(
        functools.partial(_spmm_kernel, F),
        mesh=_mesh(),
        out_type=jax.ShapeDtypeStruct((2, N_PAD, F), jnp.float32),
        scratch_types=[
            pltpu.VMEM((CHUNK,), jnp.int32),
            pltpu.VMEM((CHUNK,), jnp.int32),
            pltpu.VMEM((CHUNK, F), jnp.float32),
            pltpu.VMEM_SHARED((N_PAD, F), jnp.float32),
            pltpu.SemaphoreType.DMA,
        ],
    )


M_BLK = 1000
GRID = N_NODES // M_BLK


def _layer1_body(x_ref, a_ref, w_ref, deg_ref, g_ref, dinv_ref):
    xv = x_ref[...]
    av = a_ref[...]
    d = xv - av
    h = (jnp.dot(d, w_ref[0:128, :], preferred_element_type=jnp.float32)
         + jnp.dot(av, w_ref[128:256, :], preferred_element_type=jnp.float32))
    deg = deg_ref[0] + deg_ref[1] + 1.0
    dinv = lax.rsqrt(deg)
    dinv_ref[...] = dinv
    g_ref[...] = h * dinv


def _layer2_body(s_ref, g_ref, dinv_ref, b_ref, w_ref, g2_ref):
    s = s_ref[0] + s_ref[1]
    dinv = dinv_ref[...]
    pre = (s + g_ref[...]) * dinv + b_ref[...]
    z = jnp.maximum(pre, 0.0)
    h2 = jnp.dot(z, w_ref[...], preferred_element_type=jnp.float32)
    g2_ref[...] = jnp.concatenate(
        [h2 * dinv, jnp.zeros((h2.shape[0], 64), jnp.float32)], axis=1)


def _final_body(s_ref, g_ref, dinv_ref, b_ref, out_ref):
    s = s_ref[0] + s_ref[1]
    tot = (s + g_ref[...])[:, 0:64]
    out_ref[...] = tot * dinv_ref[...] + b_ref[...]


def kernel(x, edge_index, labels, inferenz_anchors, W1, b1, W2, b2):
    src = edge_index[0].astype(jnp.int32)
    dst = edge_index[1].astype(jnp.int32)
    pad = E_PAD - src.shape[0]
    src = jnp.concatenate([src, jnp.zeros((pad,), jnp.int32)])
    dst = jnp.concatenate([dst, jnp.full((pad,), N_NODES, jnp.int32)])

    deg_parts = _make_deg()(dst)
    deg3 = deg_parts.reshape(2, N_PAD, 1)

    g1, dinv = pl.pallas_call(
        _layer1_body,
        grid=(GRID,),
        in_specs=[
            pl.BlockSpec((M_BLK, 128), lambda m: (m, 0)),
            pl.BlockSpec((M_BLK, 128), lambda m: (m, 0)),
            pl.BlockSpec((256, 128), lambda m: (0, 0)),
            pl.BlockSpec((2, M_BLK, 1), lambda m: (0, m, 0)),
        ],
        out_specs=[
            pl.BlockSpec((M_BLK, 128), lambda m: (m, 0)),
            pl.BlockSpec((M_BLK, 1), lambda m: (m, 0)),
        ],
        out_shape=[
            jax.ShapeDtypeStruct((N_NODES, 128), jnp.float32),
            jax.ShapeDtypeStruct((N_NODES, 1), jnp.float32),
        ],
    )(x, inferenz_anchors, W1, deg3)

    s1 = _make_spmm(128)(g1, src, dst)

    g2 = pl.pallas_call(
        _layer2_body,
        grid=(GRID,),
        in_specs=[
            pl.BlockSpec((2, M_BLK, 128), lambda m: (0, m, 0)),
            pl.BlockSpec((M_BLK, 128), lambda m: (m, 0)),
            pl.BlockSpec((M_BLK, 1), lambda m: (m, 0)),
            pl.BlockSpec((1, 128), lambda m: (0, 0)),
            pl.BlockSpec((128, 64), lambda m: (0, 0)),
        ],
        out_specs=pl.BlockSpec((M_BLK, 128), lambda m: (m, 0)),
        out_shape=jax.ShapeDtypeStruct((N_NODES, 128), jnp.float32),
    )(s1, g1, dinv, b1.reshape(1, 128), W2)

    s2 = _make_spmm(128)(g2, src, dst)

    out = pl.pallas_call(
        _final_body,
        grid=(GRID,),
        in_specs=[
            pl.BlockSpec((2, M_BLK, 128), lambda m: (0, m, 0)),
            pl.BlockSpec((M_BLK, 128), lambda m: (m, 0)),
            pl.BlockSpec((M_BLK, 1), lambda m: (m, 0)),
            pl.BlockSpec((1, 64), lambda m: (0, 0)),
        ],
        out_specs=pl.BlockSpec((M_BLK, 64), lambda m: (m, 0)),
        out_shape=jax.ShapeDtypeStruct((N_NODES, 64), jnp.float32),
    )(s2, g2, dinv, b2.reshape(1, 64))

    return out
